# Initial kernel scaffold; baseline (speedup 1.0000x reference)
import jax
import jax.numpy as jnp
from jax.experimental import pallas as pl


def kernel(inputs, target_table, context_table, W, b):
    raise NotImplementedError("write your pallas kernel here")


# speedup vs baseline: 1.5376x; 1.5376x over previous
import functools

import jax
import jax.numpy as jnp
from jax import lax
from jax.experimental import pallas as pl
from jax.experimental.pallas import tpu as pltpu
from jax.experimental.pallas import tpu_sc as plsc

VOCAB = 100000
EMB = 128
BATCH = 16384

NC = 2
NS = 16
L = 16
NW = NC * NS
BPW = BATCH // NW
CHUNK = 128
NCHUNK = BPW // CHUNK
NACC = 4


def _dot_accum(buf, rows, wb_v, w_off, accs):
    def body(i, accs):
        accs = list(accs)
        d0 = i * L
        w_vec = wb_v[pl.ds(w_off + d0, L)]
        for dd in range(L):
            col = jnp.full((L,), d0 + dd, dtype=jnp.int32)
            v = plsc.load_gather(buf, [rows, col])
            accs[dd % NACC] = accs[dd % NACC] + v * w_vec[dd]
        return tuple(accs)

    return lax.fori_loop(0, EMB // L, body, accs, unroll=False)


def _body(t_idx_hbm, c_idx_hbm, tt_hbm, ct_hbm, wb_hbm, out_hbm,
          t_idx_v, c_idx_v, wb_v, t_buf, c_buf, out_v,
          sem_t0, sem_t1, sem_c0, sem_c1):
    wid = lax.axis_index("s") * NC + lax.axis_index("c")
    base = wid * BPW

    pltpu.sync_copy(wb_hbm, wb_v)
    pltpu.sync_copy(t_idx_hbm.at[pl.ds(base, BPW)], t_idx_v)
    pltpu.sync_copy(c_idx_hbm.at[pl.ds(base, BPW)], c_idx_v)

    sems = [(sem_t0, sem_c0), (sem_t1, sem_c1)]

    def start(c):
        s = c % 2
        ht = pltpu.async_copy(tt_hbm.at[t_idx_v.at[pl.ds(c * CHUNK, CHUNK)]],
                              t_buf.at[s], sems[s][0])
        hc = pltpu.async_copy(ct_hbm.at[c_idx_v.at[pl.ds(c * CHUNK, CHUNK)]],
                              c_buf.at[s], sems[s][1])
        return ht, hc

    pending = {0: start(0)}
    b_s = wb_v[pl.ds(2 * EMB, L)][0]

    for c in range(NCHUNK):
        if c + 1 < NCHUNK:
            pending[c + 1] = start(c + 1)
        ht, hc = pending.pop(c)
        ht.wait()
        hc.wait()
        s = c % 2
        for g in range(CHUNK // L):
            rows = g * L + lax.iota(jnp.int32, L)
            accs = tuple(jnp.zeros((L,), jnp.float32) for _ in range(NACC))
            accs = _dot_accum(t_buf.at[s], rows, wb_v, 0, accs)
            accs = _dot_accum(c_buf.at[s], rows, wb_v, EMB, accs)
            x = (accs[0] + accs[1]) + (accs[2] + accs[3]) + b_s
            res = 1.0 / (1.0 + jnp.exp(-x))
            out_v[pl.ds(c * CHUNK + g * L, L)] = res

    pltpu.sync_copy(out_v, out_hbm.at[pl.ds(base, BPW)])


@jax.jit
def _run(t_idx, c_idx, target_table, context_table, wb):
    mesh = plsc.VectorSubcoreMesh(core_axis_name="c", subcore_axis_name="s")
    f = pl.kernel(
        _body,
        mesh=mesh,
        compiler_params=pltpu.CompilerParams(needs_layout_passes=False),
        out_type=jax.ShapeDtypeStruct((BATCH,), jnp.float32),
        scratch_types=[
            pltpu.VMEM((BPW,), jnp.int32),
            pltpu.VMEM((BPW,), jnp.int32),
            pltpu.VMEM((2 * EMB + L,), jnp.float32),
            pltpu.VMEM((2, CHUNK, EMB), jnp.float32),
            pltpu.VMEM((2, CHUNK, EMB), jnp.float32),
            pltpu.VMEM((BPW,), jnp.float32),
            pltpu.SemaphoreType.DMA,
            pltpu.SemaphoreType.DMA,
            pltpu.SemaphoreType.DMA,
            pltpu.SemaphoreType.DMA,
        ],
    )
    return f(t_idx, c_idx, target_table, context_table, wb)


def kernel(inputs, target_table, context_table, W, b):
    idx = inputs.astype(jnp.int32)
    t_idx = idx[:, 0]
    c_idx = idx[:, 1]
    wb = jnp.concatenate([W.reshape(-1), b,
                          jnp.zeros((L - 1,), jnp.float32)])
    out = _run(t_idx, c_idx, target_table, context_table, wb)
    return out.reshape(BATCH, 1)


# speedup vs baseline: 3.6922x; 2.4013x over previous
import functools

import jax
import jax.numpy as jnp
from jax import lax
from jax.experimental import pallas as pl
from jax.experimental.pallas import tpu as pltpu
from jax.experimental.pallas import tpu_sc as plsc

VOCAB = 100000
EMB = 128
BATCH = 16384

NC = 2
NS = 16
L = 16
NW = NC * NS
BPW = BATCH // NW
CHUNK = 128
NCHUNK = BPW // CHUNK
NACC = 4


def _chunk_compute(t_buf, c_buf, w_t, w_c, b_s, lane, tr_buf, out_v, c_base):
    lane16 = lane * L

    def gbody(g, carry):
        r0 = g * L
        for rr in range(L):
            r = r0 + rr
            accs = [jnp.zeros((L,), jnp.float32) for _ in range(NACC)]
            for k in range(EMB // L):
                vt = t_buf[r, pl.ds(k * L, L)]
                accs[k % NACC] = accs[k % NACC] + vt * w_t[k]
            for k in range(EMB // L):
                vc = c_buf[r, pl.ds(k * L, L)]
                accs[(k + 2) % NACC] = accs[(k + 2) % NACC] + vc * w_c[k]
            part = (accs[0] + accs[1]) + (accs[2] + accs[3])
            plsc.store_scatter(tr_buf, [lane16 + rr], part)
        sums = [tr_buf[pl.ds(l * L, L)] for l in range(0, L, NACC)]
        for l in range(L):
            if l % NACC:
                sums[l // NACC] = sums[l // NACC] + tr_buf[pl.ds(l * L, L)]
        x = (sums[0] + sums[1]) + (sums[2] + sums[3]) + b_s
        out_v[pl.ds(c_base + g * L, L)] = 1.0 / (1.0 + jnp.exp(-x))
        return carry

    lax.fori_loop(0, CHUNK // L, gbody, 0, unroll=False)


def _body(t_idx_hbm, c_idx_hbm, tt_hbm, ct_hbm, wb_hbm, out_hbm,
          t_idx_v, c_idx_v, wb_v, t_buf, c_buf, tr_buf, out_v,
          sem_t0, sem_t1, sem_c0, sem_c1):
    wid = lax.axis_index("s") * NC + lax.axis_index("c")
    base = wid * BPW

    pltpu.sync_copy(wb_hbm, wb_v)
    pltpu.sync_copy(t_idx_hbm.at[pl.ds(base, BPW)], t_idx_v)
    pltpu.sync_copy(c_idx_hbm.at[pl.ds(base, BPW)], c_idx_v)

    sems = [(sem_t0, sem_c0), (sem_t1, sem_c1)]

    def start(c):
        s = c % 2
        ht = pltpu.async_copy(tt_hbm.at[t_idx_v.at[pl.ds(c * CHUNK, CHUNK)]],
                              t_buf.at[s], sems[s][0])
        hc = pltpu.async_copy(ct_hbm.at[c_idx_v.at[pl.ds(c * CHUNK, CHUNK)]],
                              c_buf.at[s], sems[s][1])
        return ht, hc

    pending = {0: start(0)}
    b_s = wb_v[pl.ds(2 * EMB, L)][0]
    w_t = [wb_v[pl.ds(k * L, L)] for k in range(EMB // L)]
    w_c = [wb_v[pl.ds(EMB + k * L, L)] for k in range(EMB // L)]
    lane = lax.iota(jnp.int32, L)

    for c in range(NCHUNK):
        if c + 1 < NCHUNK:
            pending[c + 1] = start(c + 1)
        ht, hc = pending.pop(c)
        ht.wait()
        hc.wait()
        s = c % 2
        _chunk_compute(t_buf.at[s], c_buf.at[s], w_t, w_c, b_s, lane,
                       tr_buf, out_v, c * CHUNK)

    pltpu.sync_copy(out_v, out_hbm.at[pl.ds(base, BPW)])


@jax.jit
def _run(t_idx, c_idx, target_table, context_table, wb):
    mesh = plsc.VectorSubcoreMesh(core_axis_name="c", subcore_axis_name="s")
    f = pl.kernel(
        _body,
        mesh=mesh,
        compiler_params=pltpu.CompilerParams(needs_layout_passes=False),
        out_type=jax.ShapeDtypeStruct((BATCH,), jnp.float32),
        scratch_types=[
            pltpu.VMEM((BPW,), jnp.int32),
            pltpu.VMEM((BPW,), jnp.int32),
            pltpu.VMEM((2 * EMB + L,), jnp.float32),
            pltpu.VMEM((2, CHUNK, EMB), jnp.float32),
            pltpu.VMEM((2, CHUNK, EMB), jnp.float32),
            pltpu.VMEM((L * L,), jnp.float32),
            pltpu.VMEM((BPW,), jnp.float32),
            pltpu.SemaphoreType.DMA,
            pltpu.SemaphoreType.DMA,
            pltpu.SemaphoreType.DMA,
            pltpu.SemaphoreType.DMA,
        ],
    )
    return f(t_idx, c_idx, target_table, context_table, wb)


def kernel(inputs, target_table, context_table, W, b):
    idx = inputs.astype(jnp.int32)
    t_idx = idx[:, 0]
    c_idx = idx[:, 1]
    wb = jnp.concatenate([W.reshape(-1), b,
                          jnp.zeros((L - 1,), jnp.float32)])
    out = _run(t_idx, c_idx, target_table, context_table, wb)
    return out.reshape(BATCH, 1)
